# baseline (speedup 1.0000x reference)
import functools

import jax
import jax.numpy as jnp
from jax import lax
from jax.experimental import pallas as pl
from jax.experimental.pallas import tpu as pltpu
from jax.experimental.pallas import tpu_sc as plsc

N = 10000
E = 160000
D = 256
H = 128
NPAD = 10240
ROWS_PER_TEC = NPAD // 16
CHUNK = 320
NR = 32
EPT = CHUNK * NR
E_PAD = EPT * 16
WCHUNK = 80


def _mm_body(lo_ref, hi_ref, w_ref, out_ref):
    xblk = jnp.concatenate([lo_ref[...], hi_ref[...]], axis=1)
    out_ref[...] = jnp.dot(xblk, w_ref[...], preferred_element_type=jnp.float32)


def _matmul(lo, hi, w):
    blk = 1000
    grid = (N // blk, 2)
    return pl.pallas_call(
        _mm_body,
        grid=grid,
        in_specs=[
            pl.BlockSpec((blk, H), lambda i, j: (i, 0)),
            pl.BlockSpec((blk, H), lambda i, j: (i, 0)),
            pl.BlockSpec((D, H), lambda i, j: (0, j)),
        ],
        out_specs=pl.BlockSpec((blk, H), lambda i, j: (j * (N // blk) + i, 0)),
        out_shape=jax.ShapeDtypeStruct((2 * N, H), jnp.float32),
    )(lo, hi, w)


def _zero16():
    return jnp.zeros((16,), jnp.float32)


_GDN = lax.GatherDimensionNumbers(
    offset_dims=(), collapsed_slice_dims=(0,), start_index_map=(0,))


def _splat(vec, lane):
    idx = jnp.full((16, 1), lane, jnp.int32)
    return lax.gather(vec, idx, _GDN, slice_sizes=(1,),
                      mode=lax.GatherScatterMode.PROMISE_IN_BOUNDS)


def _prop_body(apply_relu,
               table, packed_hbm, bias_hbm, out_hbm,
               rowsb, ebig0, ebig1, sdst, sexpl, bias_v, denom_v,
               isem0, isem1, gsem, ssem,
               acc_sh, denom_sh):
    c = lax.axis_index("c")
    s = lax.axis_index("s")
    ebig = (ebig0, ebig1)
    isem = (isem0, isem1)

    rowbase = s * ROWS_PER_TEC
    coff = c * N

    def zrow(i, _):
        for j in range(H // 16):
            rowsb[i, pl.ds(16 * j, 16)] = _zero16()
        return 0

    lax.fori_loop(0, CHUNK, zrow, 0)

    def zden(i, _):
        denom_v[pl.ds(16 * i, 16)] = _zero16()
        return 0

    lax.fori_loop(0, ROWS_PER_TEC // 16, zden, 0)

    for q in range(4):
        pltpu.sync_copy(rowsb.at[pl.ds(0, 160)],
                        acc_sh.at[pl.ds(rowbase + q * 160, 160)])
    pltpu.sync_copy(denom_v, denom_sh.at[pl.ds(rowbase, ROWS_PER_TEC)])
    pltpu.sync_copy(bias_hbm.at[pl.ds(c * H, H)], bias_v)

    plsc.subcore_barrier()

    def issue_idx(r, p):
        pltpu.async_copy(packed_hbm.at[s * NR + r], ebig[p], isem[p])

    def wait_idx(r, p):
        pltpu.make_async_copy(packed_hbm.at[s * NR + r], ebig[p], isem[p]).wait()

    def prep(p):
        coffv = jnp.full((16,), coff, jnp.int32)

        def pstep(m, _):
            ebig[p][pl.ds(16 * m, 16)] = ebig[p][pl.ds(16 * m, 16)] + coffv
            sdst[pl.ds(16 * m, 16)] = ebig[p][pl.ds(CHUNK + 16 * m, 16)]
            lb = lax.bitcast_convert_type(
                ebig[p][pl.ds(2 * CHUNK + 16 * m, 16)], jnp.float32)
            sexpl[pl.ds(16 * m, 16)] = jnp.exp(lb)
            return 0

        lax.fori_loop(0, CHUNK // 16, pstep, 0)

    def scale():
        def sstep(m, _):
            ev = sexpl[pl.ds(16 * m, 16)]
            for jj in range(16):
                spl = _splat(ev, jj)
                row = rowsb.at[16 * m + jj]
                for j in range(H // 16):
                    row[pl.ds(16 * j, 16)] = row[pl.ds(16 * j, 16)] * spl
            return 0

        lax.fori_loop(0, CHUNK // 16, sstep, 0)

    def drain_scatter():
        pltpu.make_async_copy(rowsb, acc_sh.at[sdst], ssem).wait()
        pltpu.make_async_copy(sexpl, denom_sh.at[sdst], ssem).wait()

    def round_fn(r, p, q):
        wait_idx(r, p)

        @pl.when(r + 1 < NR)
        def _():
            issue_idx(r + 1, q)

        @pl.when(r >= 1)
        def _():
            drain_scatter()

        prep(p)
        pltpu.async_copy(table.at[ebig[p].at[pl.ds(0, CHUNK)]], rowsb,
                         gsem).wait()
        scale()
        pltpu.async_copy(rowsb, acc_sh.at[sdst], ssem, add=True)
        pltpu.async_copy(sexpl, denom_sh.at[sdst], ssem, add=True)

    issue_idx(0, 0)

    def pair(i, _):
        round_fn(2 * i, 0, 1)
        round_fn(2 * i + 1, 1, 0)
        return 0

    lax.fori_loop(0, NR // 2, pair, 0)
    drain_scatter()

    plsc.subcore_barrier()

    pltpu.sync_copy(denom_sh.at[pl.ds(rowbase, ROWS_PER_TEC)], denom_v)

    def wchunk(cc, _):
        base = rowbase + cc * WCHUNK

        @pl.when(base < N)
        def _():
            pltpu.sync_copy(acc_sh.at[pl.ds(base, WCHUNK)],
                            rowsb.at[pl.ds(0, WCHUNK)])

            def node(m, _):
                dv = denom_v[pl.ds(cc * WCHUNK + 16 * m, 16)] + jnp.full(
                    (16,), 1e-16, jnp.float32)
                for jj in range(16):
                    dspl = _splat(dv, jj)
                    row = rowsb.at[16 * m + jj]
                    for j in range(H // 16):
                        v = (row[pl.ds(16 * j, 16)] / dspl
                             + bias_v[pl.ds(16 * j, 16)])
                        if apply_relu:
                            v = jnp.maximum(v, jnp.zeros((16,), jnp.float32))
                        row[pl.ds(16 * j, 16)] = v
                return 0

            lax.fori_loop(0, WCHUNK // 16, node, 0)
            pltpu.sync_copy(rowsb.at[pl.ds(0, WCHUNK)],
                            out_hbm.at[c, pl.ds(base, WCHUNK)])

        return 0

    lax.fori_loop(0, ROWS_PER_TEC // WCHUNK, wchunk, 0)


def _prop(table, packed, bias, apply_relu):
    mesh = plsc.VectorSubcoreMesh(core_axis_name="c", subcore_axis_name="s")
    kfn = pl.kernel(
        functools.partial(_prop_body, apply_relu),
        out_type=jax.ShapeDtypeStruct((2, N, H), jnp.float32),
        mesh=mesh,
        scratch_types=[
            pltpu.VMEM((CHUNK, H), jnp.float32),
            pltpu.VMEM((3 * CHUNK,), jnp.int32),
            pltpu.VMEM((3 * CHUNK,), jnp.int32),
            pltpu.VMEM((CHUNK,), jnp.int32),
            pltpu.VMEM((CHUNK,), jnp.float32),
            pltpu.VMEM((H,), jnp.float32),
            pltpu.VMEM((ROWS_PER_TEC,), jnp.float32),
            pltpu.SemaphoreType.DMA,
            pltpu.SemaphoreType.DMA,
            pltpu.SemaphoreType.DMA,
            pltpu.SemaphoreType.DMA,
            pltpu.VMEM_SHARED((NPAD, H), jnp.float32),
            pltpu.VMEM_SHARED((NPAD,), jnp.float32),
        ],
        name="gcn_prop",
    )
    return kfn(table, packed, bias)


def _pack_edges(edge_index, logits):
    pad = E_PAD - E
    src = jnp.concatenate([edge_index[0], jnp.zeros((pad,), jnp.int32)])
    dst = jnp.concatenate([edge_index[1], jnp.zeros((pad,), jnp.int32)])
    lbits = lax.bitcast_convert_type(
        jnp.concatenate([logits, jnp.full((pad,), -1e30, jnp.float32)]),
        jnp.int32)
    stacked = jnp.stack([src.reshape(16, NR, CHUNK),
                         dst.reshape(16, NR, CHUNK),
                         lbits.reshape(16, NR, CHUNK)], axis=2)
    return stacked.reshape(16 * NR, 3 * CHUNK)


def kernel(x, edge_index, edge_weight_logits, W1, b1, W2, b2):
    packed = _pack_edges(edge_index, edge_weight_logits)
    h1 = _matmul(x[:, :H], x[:, H:], W1)
    o1 = _prop(h1, packed, b1, apply_relu=True)
    h2 = _matmul(o1[0], o1[1], W2)
    o2 = _prop(h2, packed, b2, apply_relu=False)
    return jnp.concatenate([o2[0], o2[1]], axis=1)[None]


# speedup vs baseline: 1.5048x; 1.5048x over previous
import functools

import jax
import jax.numpy as jnp
from jax import lax
from jax.experimental import pallas as pl
from jax.experimental.pallas import tpu as pltpu
from jax.experimental.pallas import tpu_sc as plsc

N = 10000
E = 160000
D = 256
H = 128
NPAD = 10240
ROWS_PER_TEC = NPAD // 16
EDGES_PER_TEC = E // 16
CHUNK = 320
NCHUNKS = EDGES_PER_TEC // CHUNK
TAIL = EDGES_PER_TEC - NCHUNKS * CHUNK


def _mm_body(lo_ref, hi_ref, w_ref, out_ref):
    xblk = jnp.concatenate([lo_ref[...], hi_ref[...]], axis=1)
    out_ref[...] = jnp.dot(xblk, w_ref[...], preferred_element_type=jnp.float32)


def _matmul(lo, hi, w):
    blk = 1000
    grid = (N // blk, 2)
    return pl.pallas_call(
        _mm_body,
        grid=grid,
        in_specs=[
            pl.BlockSpec((blk, H), lambda i, j: (i, 0)),
            pl.BlockSpec((blk, H), lambda i, j: (i, 0)),
            pl.BlockSpec((D, H), lambda i, j: (0, j)),
        ],
        out_specs=pl.BlockSpec((blk, H), lambda i, j: (j * (N // blk) + i, 0)),
        out_shape=jax.ShapeDtypeStruct((2 * N, H), jnp.float32),
    )(lo, hi, w)


def _zero16():
    return jnp.zeros((16,), jnp.float32)


_GDN = lax.GatherDimensionNumbers(
    offset_dims=(), collapsed_slice_dims=(0,), start_index_map=(0,))


def _splat(vec, lane):
    idx = jnp.full((16, 1), lane, jnp.int32)
    return lax.gather(vec, idx, _GDN, slice_sizes=(1,),
                      mode=lax.GatherScatterMode.PROMISE_IN_BOUNDS)


def _prop_body(apply_relu,
               table, src_hbm, dst_hbm, log_hbm, bias_hbm, out_hbm,
               rows_v, src_v, dst_v, expl_v, bias_v, denom_v, sem,
               acc_sh, denom_sh):
    c = lax.axis_index("c")
    s = lax.axis_index("s")

    rowbase = s * ROWS_PER_TEC

    def zrow(i, _):
        for j in range(H // 16):
            rows_v[i, pl.ds(16 * j, 16)] = _zero16()
        return 0

    lax.fori_loop(0, CHUNK, zrow, 0)

    def zden(i, _):
        denom_v[pl.ds(16 * i, 16)] = _zero16()
        return 0

    lax.fori_loop(0, ROWS_PER_TEC // 16, zden, 0)

    pltpu.sync_copy(rows_v, acc_sh.at[pl.ds(rowbase, CHUNK)])
    pltpu.sync_copy(rows_v.at[pl.ds(0, ROWS_PER_TEC - CHUNK)],
                    acc_sh.at[pl.ds(rowbase + CHUNK, ROWS_PER_TEC - CHUNK)])
    pltpu.sync_copy(denom_v, denom_sh.at[pl.ds(rowbase, ROWS_PER_TEC)])

    pltpu.sync_copy(bias_hbm, bias_v)

    plsc.subcore_barrier()

    ebase = s * EDGES_PER_TEC
    coff = c * N

    def edge_round(nb, b):
        pltpu.sync_copy(src_hbm.at[pl.ds(b, nb)], src_v.at[pl.ds(0, nb)])
        pltpu.sync_copy(dst_hbm.at[pl.ds(b, nb)], dst_v.at[pl.ds(0, nb)])
        pltpu.sync_copy(log_hbm.at[pl.ds(b, nb)], expl_v.at[pl.ds(0, nb)])

        coffv = jnp.full((16,), coff, jnp.int32)

        def prep(m, _):
            src_v[pl.ds(16 * m, 16)] = src_v[pl.ds(16 * m, 16)] + coffv
            expl_v[pl.ds(16 * m, 16)] = jnp.exp(expl_v[pl.ds(16 * m, 16)])
            return 0

        lax.fori_loop(0, nb // 16, prep, 0)

        pltpu.async_copy(table.at[src_v.at[pl.ds(0, nb)]],
                         rows_v.at[pl.ds(0, nb)], sem).wait()

        def scale(m, _):
            ev = expl_v[pl.ds(16 * m, 16)]
            for jj in range(16):
                spl = _splat(ev, jj)
                row = rows_v.at[16 * m + jj]
                for j in range(H // 16):
                    row[pl.ds(16 * j, 16)] = row[pl.ds(16 * j, 16)] * spl
            return 0

        lax.fori_loop(0, nb // 16, scale, 0)

        pltpu.sync_copy(rows_v.at[pl.ds(0, nb)],
                        acc_sh.at[dst_v.at[pl.ds(0, nb)]], add=True)
        pltpu.sync_copy(expl_v.at[pl.ds(0, nb)],
                        denom_sh.at[dst_v.at[pl.ds(0, nb)]], add=True)

    def full_round(k, _):
        edge_round(CHUNK, ebase + k * CHUNK)
        return 0

    lax.fori_loop(0, NCHUNKS, full_round, 0)
    edge_round(TAIL, ebase + NCHUNKS * CHUNK)

    plsc.subcore_barrier()

    pltpu.sync_copy(denom_sh.at[pl.ds(rowbase, ROWS_PER_TEC)], denom_v)

    def write_chunk(start, nrows):
        pltpu.sync_copy(acc_sh.at[pl.ds(rowbase + start, nrows)],
                        rows_v.at[pl.ds(0, nrows)])

        def node(m, _):
            dv = denom_v[pl.ds(start + 16 * m, 16)] + jnp.full((16,), 1e-16,
                                                              jnp.float32)
            for jj in range(16):
                dspl = _splat(dv, jj)
                row = rows_v.at[16 * m + jj]
                for j in range(H // 16):
                    v = (row[pl.ds(16 * j, 16)] / dspl
                         + bias_v[pl.ds(c * H + 16 * j, 16)])
                    if apply_relu:
                        v = jnp.maximum(v, jnp.zeros((16,), jnp.float32))
                    row[pl.ds(16 * j, 16)] = v
            return 0

        lax.fori_loop(0, nrows // 16, node, 0)
        pltpu.sync_copy(rows_v.at[pl.ds(0, nrows)],
                        out_hbm.at[c, pl.ds(rowbase + start, nrows)])

    write_chunk(0, CHUNK)

    @pl.when(rowbase + 2 * CHUNK <= N)
    def _():
        write_chunk(CHUNK, CHUNK)

    @pl.when(rowbase + 2 * CHUNK > N)
    def _():
        write_chunk(CHUNK, N - 15 * ROWS_PER_TEC - CHUNK)


def _prop(table, src, dst, logits, bias, apply_relu):
    mesh = plsc.VectorSubcoreMesh(core_axis_name="c", subcore_axis_name="s")
    kfn = pl.kernel(
        functools.partial(_prop_body, apply_relu),
        out_type=jax.ShapeDtypeStruct((2, N, H), jnp.float32),
        mesh=mesh,
        scratch_types=[
            pltpu.VMEM((CHUNK, H), jnp.float32),
            pltpu.VMEM((CHUNK,), jnp.int32),
            pltpu.VMEM((CHUNK,), jnp.int32),
            pltpu.VMEM((CHUNK,), jnp.float32),
            pltpu.VMEM((2 * H,), jnp.float32),
            pltpu.VMEM((ROWS_PER_TEC,), jnp.float32),
            pltpu.SemaphoreType.DMA,
            pltpu.VMEM_SHARED((NPAD, H), jnp.float32),
            pltpu.VMEM_SHARED((NPAD,), jnp.float32),
        ],
        name="gcn_prop",
    )
    return kfn(table, src, dst, logits, bias)


def kernel(x, edge_index, edge_weight_logits, W1, b1, W2, b2):
    src = edge_index[0]
    dst = edge_index[1]
    h1 = _matmul(x[:, :H], x[:, H:], W1)
    o1 = _prop(h1, src, dst, edge_weight_logits, b1, apply_relu=True)
    h2 = _matmul(o1[0], o1[1], W2)
    o2 = _prop(h2, src, dst, edge_weight_logits, b2, apply_relu=False)
    return jnp.concatenate([o2[0], o2[1]], axis=1)[None]


# speedup vs baseline: 1.6665x; 1.1074x over previous
import functools

import jax
import jax.numpy as jnp
from jax import lax
from jax.experimental import pallas as pl
from jax.experimental.pallas import tpu as pltpu
from jax.experimental.pallas import tpu_sc as plsc

N = 10000
E = 160000
D = 256
H = 128
NPAD = 10240
ROWS_PER_TEC = NPAD // 16
EDGES_PER_TEC = E // 16
CHUNK = 320
NCHUNKS = EDGES_PER_TEC // CHUNK
TAIL = EDGES_PER_TEC - NCHUNKS * CHUNK


def _mm_body(lo_ref, hi_ref, w_ref, out_ref):
    xblk = jnp.concatenate([lo_ref[...], hi_ref[...]], axis=1)
    out_ref[...] = jnp.dot(xblk, w_ref[...], preferred_element_type=jnp.float32)


def _matmul(lo, hi, w):
    blk = 1000
    grid = (N // blk, 2)
    return pl.pallas_call(
        _mm_body,
        grid=grid,
        in_specs=[
            pl.BlockSpec((blk, H), lambda i, j: (i, 0)),
            pl.BlockSpec((blk, H), lambda i, j: (i, 0)),
            pl.BlockSpec((D, H), lambda i, j: (0, j)),
        ],
        out_specs=pl.BlockSpec((blk, H), lambda i, j: (j * (N // blk) + i, 0)),
        out_shape=jax.ShapeDtypeStruct((2 * N, H), jnp.float32),
    )(lo, hi, w)


def _zero16():
    return jnp.zeros((16,), jnp.float32)


_GDN = lax.GatherDimensionNumbers(
    offset_dims=(), collapsed_slice_dims=(0,), start_index_map=(0,))


def _splat(vec, lane):
    idx = jnp.full((16, 1), lane, jnp.int32)
    return lax.gather(vec, idx, _GDN, slice_sizes=(1,),
                      mode=lax.GatherScatterMode.PROMISE_IN_BOUNDS)


def _prop_body(apply_relu,
               table, src_hbm, dst_hbm, log_hbm, bias_hbm, out_hbm,
               rows_v, src_v, dst_v, expl_v, bias_v, denom_v, sem,
               acc_sh, denom_sh):
    c = lax.axis_index("c")
    s = lax.axis_index("s")

    rowbase = s * ROWS_PER_TEC

    def zrow(i, _):
        for j in range(H // 16):
            rows_v[i, pl.ds(16 * j, 16)] = _zero16()
        return 0

    lax.fori_loop(0, CHUNK, zrow, 0)

    def zden(i, _):
        denom_v[pl.ds(16 * i, 16)] = _zero16()
        return 0

    lax.fori_loop(0, ROWS_PER_TEC // 16, zden, 0)

    pltpu.sync_copy(rows_v, acc_sh.at[pl.ds(rowbase, CHUNK)])
    pltpu.sync_copy(rows_v.at[pl.ds(0, ROWS_PER_TEC - CHUNK)],
                    acc_sh.at[pl.ds(rowbase + CHUNK, ROWS_PER_TEC - CHUNK)])
    pltpu.sync_copy(denom_v, denom_sh.at[pl.ds(rowbase, ROWS_PER_TEC)])

    pltpu.sync_copy(bias_hbm, bias_v)

    plsc.subcore_barrier()

    ebase = s * EDGES_PER_TEC
    coff = c * N

    def edge_round(nb, b):
        d1 = pltpu.async_copy(src_hbm.at[pl.ds(b, nb)],
                              src_v.at[pl.ds(0, nb)], sem)
        d2 = pltpu.async_copy(dst_hbm.at[pl.ds(b, nb)],
                              dst_v.at[pl.ds(0, nb)], sem)
        d3 = pltpu.async_copy(log_hbm.at[pl.ds(b, nb)],
                              expl_v.at[pl.ds(0, nb)], sem)
        d1.wait()
        d2.wait()
        d3.wait()

        coffv = jnp.full((16,), coff, jnp.int32)

        def prep(m, _):
            src_v[pl.ds(16 * m, 16)] = src_v[pl.ds(16 * m, 16)] + coffv
            expl_v[pl.ds(16 * m, 16)] = jnp.exp(expl_v[pl.ds(16 * m, 16)])
            return 0

        lax.fori_loop(0, nb // 16, prep, 0)

        pltpu.async_copy(table.at[src_v.at[pl.ds(0, nb)]],
                         rows_v.at[pl.ds(0, nb)], sem).wait()

        def scale(m, _):
            ev = expl_v[pl.ds(16 * m, 16)]
            for jj in range(16):
                spl = _splat(ev, jj)
                row = rows_v.at[16 * m + jj]
                for j in range(H // 16):
                    row[pl.ds(16 * j, 16)] = row[pl.ds(16 * j, 16)] * spl
            return 0

        lax.fori_loop(0, nb // 16, scale, 0)

        s1 = pltpu.async_copy(rows_v.at[pl.ds(0, nb)],
                              acc_sh.at[dst_v.at[pl.ds(0, nb)]], sem, add=True)
        s2 = pltpu.async_copy(expl_v.at[pl.ds(0, nb)],
                              denom_sh.at[dst_v.at[pl.ds(0, nb)]], sem, add=True)
        s1.wait()
        s2.wait()

    def full_round(k, _):
        edge_round(CHUNK, ebase + k * CHUNK)
        return 0

    lax.fori_loop(0, NCHUNKS, full_round, 0)
    edge_round(TAIL, ebase + NCHUNKS * CHUNK)

    plsc.subcore_barrier()

    pltpu.sync_copy(denom_sh.at[pl.ds(rowbase, ROWS_PER_TEC)], denom_v)

    def write_chunk(start, nrows):
        pltpu.sync_copy(acc_sh.at[pl.ds(rowbase + start, nrows)],
                        rows_v.at[pl.ds(0, nrows)])

        def node(m, _):
            dv = denom_v[pl.ds(start + 16 * m, 16)] + jnp.full((16,), 1e-16,
                                                              jnp.float32)
            for jj in range(16):
                dspl = _splat(dv, jj)
                row = rows_v.at[16 * m + jj]
                for j in range(H // 16):
                    v = (row[pl.ds(16 * j, 16)] / dspl
                         + bias_v[pl.ds(c * H + 16 * j, 16)])
                    if apply_relu:
                        v = jnp.maximum(v, jnp.zeros((16,), jnp.float32))
                    row[pl.ds(16 * j, 16)] = v
            return 0

        lax.fori_loop(0, nrows // 16, node, 0)
        pltpu.sync_copy(rows_v.at[pl.ds(0, nrows)],
                        out_hbm.at[c, pl.ds(rowbase + start, nrows)])

    write_chunk(0, CHUNK)

    @pl.when(rowbase + 2 * CHUNK <= N)
    def _():
        write_chunk(CHUNK, CHUNK)

    @pl.when(rowbase + 2 * CHUNK > N)
    def _():
        write_chunk(CHUNK, N - 15 * ROWS_PER_TEC - CHUNK)


def _prop(table, src, dst, logits, bias, apply_relu):
    mesh = plsc.VectorSubcoreMesh(core_axis_name="c", subcore_axis_name="s")
    kfn = pl.kernel(
        functools.partial(_prop_body, apply_relu),
        out_type=jax.ShapeDtypeStruct((2, N, H), jnp.float32),
        mesh=mesh,
        scratch_types=[
            pltpu.VMEM((CHUNK, H), jnp.float32),
            pltpu.VMEM((CHUNK,), jnp.int32),
            pltpu.VMEM((CHUNK,), jnp.int32),
            pltpu.VMEM((CHUNK,), jnp.float32),
            pltpu.VMEM((2 * H,), jnp.float32),
            pltpu.VMEM((ROWS_PER_TEC,), jnp.float32),
            pltpu.SemaphoreType.DMA,
            pltpu.VMEM_SHARED((NPAD, H), jnp.float32),
            pltpu.VMEM_SHARED((NPAD,), jnp.float32),
        ],
        name="gcn_prop",
    )
    return kfn(table, src, dst, logits, bias)


def kernel(x, edge_index, edge_weight_logits, W1, b1, W2, b2):
    src = edge_index[0]
    dst = edge_index[1]
    h1 = _matmul(x[:, :H], x[:, H:], W1)
    o1 = _prop(h1, src, dst, edge_weight_logits, b1, apply_relu=True)
    h2 = _matmul(o1[0], o1[1], W2)
    o2 = _prop(h2, src, dst, edge_weight_logits, b2, apply_relu=False)
    return jnp.concatenate([o2[0], o2[1]], axis=1)[None]
